# Initial kernel scaffold; baseline (speedup 1.0000x reference)
import jax
import jax.numpy as jnp
from jax.experimental import pallas as pl


def kernel(x):
    raise NotImplementedError("write your pallas kernel here")


# speedup vs baseline: 15.4778x; 15.4778x over previous
import functools

import numpy as np
import jax
import jax.numpy as jnp
from jax import lax
from jax.experimental import pallas as pl
from jax.experimental.pallas import tpu as pltpu
from jax.experimental.pallas import tpu_sc as plsc

_MASK_BITS = (
    0x6dfda5ef, 0xf7ffb56f, 0xef5bff7f, 0x1edbead9,
    0xfdf7fdfb, 0xaeedb2eb, 0xdbe75ed7, 0x5bffff7c,
    0x7d9aef9b, 0xffbfbffd, 0xcbbfacff, 0xf7bdf6da,
    0x9b7f6dfb, 0xb5b1efbe, 0xb7cb8ebf, 0xbb60d6ff,
    0xbcbcdf7f, 0xf8ff379f, 0x3fddfbfe, 0xcf6ace7f,
    0xd8fff4df, 0xdedeeeef, 0xf7dffcfb, 0xfffdffff,
    0x7b4dffb9, 0xcd6acf7d, 0xd7dddeef, 0xfa7abffb,
    0xf7ed56df, 0xf3fcbf8b, 0x97efe3a8, 0xe3afb96f,
)

_NPLANES = 96
_NPR = 32
_PS = 16
_W = 512
_NBUF = 2
_NC = 2
_NS = 16


def _build_colmask() -> np.ndarray:
    m = np.empty((_NPR, _W), np.float32)
    for r in range(_NPR):
        for c in range(_NPR):
            keep = 0.0 if ((_MASK_BITS[r] >> c) & 1) else 1.0
            m[r, c * _PS:(c + 1) * _PS] = keep
    return m


_COLMASK = _build_colmask()


def _sc_mask_body(x_hbm, mask_hbm, out_hbm, mask_v, in_buf, out_buf,
                  in_sem, out_sem):
    wid = lax.axis_index("s") * _NC + lax.axis_index("c")

    pltpu.sync_copy(mask_hbm.at[wid], mask_v)

    def start_in(b, plane):
        pltpu.async_copy(x_hbm.at[plane, wid], in_buf.at[b], in_sem.at[b])

    def wait_in(b):
        pltpu.make_async_copy(x_hbm.at[0, 0], in_buf.at[b],
                              in_sem.at[b]).wait()

    def start_out(b, plane):
        pltpu.async_copy(out_buf.at[b], out_hbm.at[plane, wid],
                         out_sem.at[b])

    def wait_out(b):
        pltpu.make_async_copy(out_buf.at[b], out_hbm.at[0, 0],
                              out_sem.at[b]).wait()

    for b in range(_NBUF):
        start_in(b, b)

    n_groups = _NPLANES // _NBUF

    def step(g, carry):
        for b in range(_NBUF):
            plane = g * _NBUF + b

            @pl.when(g >= 1)
            def _drain_prev_out(b=b):
                wait_out(b)

            wait_in(b)
            for c in range(_NPR):
                m = mask_v[pl.ds(c * _PS, _PS)]
                for r in range(_PS):
                    sl = pl.ds(c * _PS, _PS)
                    out_buf[b, r, sl] = in_buf[b, r, sl] * m
            start_out(b, plane)

            @pl.when(g <= n_groups - 2)
            def _prefetch_next(b=b, plane=plane):
                start_in(b, plane + _NBUF)
        return carry

    lax.fori_loop(0, n_groups, step, 0)

    for b in range(_NBUF):
        wait_out(b)


@functools.partial(jax.jit, static_argnums=())
def _masked(x4, colmask):
    call = pl.kernel(
        _sc_mask_body,
        out_type=jax.ShapeDtypeStruct((_NPLANES, _NPR, _PS, _W),
                                      jnp.float32),
        mesh=plsc.VectorSubcoreMesh(core_axis_name="c",
                                    subcore_axis_name="s",
                                    num_cores=_NC, num_subcores=_NS),
        scratch_types=[
            pltpu.VMEM((_W,), jnp.float32),
            pltpu.VMEM((_NBUF, _PS, _W), jnp.float32),
            pltpu.VMEM((_NBUF, _PS, _W), jnp.float32),
            pltpu.SemaphoreType.DMA((_NBUF,)),
            pltpu.SemaphoreType.DMA((_NBUF,)),
        ],
    )
    return call(x4, colmask)


def kernel(x):
    x4 = x.reshape(_NPLANES, _NPR, _PS, _W)
    out4 = _masked(x4, jnp.asarray(_COLMASK))
    return out4.reshape(32, 3, 512, 512)
